# Initial kernel scaffold; baseline (speedup 1.0000x reference)
import jax
import jax.numpy as jnp
from jax.experimental import pallas as pl


def kernel(inputs, hx, src0, dst0, w0, src1, dst1, w1, Wg, bg, Wc, bc):
    raise NotImplementedError("write your pallas kernel here")


# speedup vs baseline: 1.2127x; 1.2127x over previous
import functools

import jax
import jax.numpy as jnp
from jax.experimental import pallas as pl
from jax.experimental.pallas import tpu as pltpu

N = 10000
E = 160000
BATCH = 8
INPUT_DIM = 2
UNITS = 64
IN_SIZE = INPUT_DIM + UNITS
NUM_MAT = 5
ROW = BATCH * IN_SIZE

BM = 800


def _spmm(src, dst, w, x):
    gathered = x[src] * w[:, None]
    return jax.ops.segment_sum(gathered, dst, num_segments=N)


def _diffusion(x0, src0, dst0, w0, src1, dst1, w1):
    m1 = _spmm(src0, dst0, w0, x0)
    m2 = 2.0 * _spmm(src0, dst0, w0, m1) - x0
    m3 = _spmm(src1, dst1, w1, m1)
    m4 = 2.0 * _spmm(src1, dst1, w1, m3) - m1
    return [x0, m1, m2, m3, m4]


def _gate_body(x_ref, w_ref, b_ref, hx_ref, state2_ref, u_ref):
    v = jnp.dot(x_ref[...], w_ref[...], preferred_element_type=jnp.float32)
    v = jax.nn.sigmoid(v + b_ref[...])
    r = v[:, :UNITS]
    u = v[:, UNITS:]
    hx = hx_ref[...]
    state2_ref[...] = r * hx
    u_ref[...] = u


def _cand_body(x_ref, w_ref, b_ref, hx_ref, u_ref, out_ref):
    v = jnp.dot(x_ref[...], w_ref[...], preferred_element_type=jnp.float32)
    c = jnp.tanh(v + b_ref[...])
    u = u_ref[...]
    out_ref[...] = u * hx_ref[...] + (1.0 - u) * c


def _gate_call(x5, Wg_r, bg, hxr):
    grid = (BATCH * N // BM,)
    return pl.pallas_call(
        _gate_body,
        grid=grid,
        in_specs=[
            pl.BlockSpec((BM, NUM_MAT * IN_SIZE), lambda i: (i, 0)),
            pl.BlockSpec((NUM_MAT * IN_SIZE, 2 * UNITS), lambda i: (0, 0)),
            pl.BlockSpec((1, 2 * UNITS), lambda i: (0, 0)),
            pl.BlockSpec((BM, UNITS), lambda i: (i, 0)),
        ],
        out_specs=[
            pl.BlockSpec((BM, UNITS), lambda i: (i, 0)),
            pl.BlockSpec((BM, UNITS), lambda i: (i, 0)),
        ],
        out_shape=[
            jax.ShapeDtypeStruct((BATCH * N, UNITS), jnp.float32),
            jax.ShapeDtypeStruct((BATCH * N, UNITS), jnp.float32),
        ],
    )(x5, Wg_r, bg.reshape(1, -1), hxr)


def _cand_call(x5, Wc_r, bc, hxr, u):
    grid = (BATCH * N // BM,)
    return pl.pallas_call(
        _cand_body,
        grid=grid,
        in_specs=[
            pl.BlockSpec((BM, NUM_MAT * IN_SIZE), lambda i: (i, 0)),
            pl.BlockSpec((NUM_MAT * IN_SIZE, UNITS), lambda i: (0, 0)),
            pl.BlockSpec((1, UNITS), lambda i: (0, 0)),
            pl.BlockSpec((BM, UNITS), lambda i: (i, 0)),
            pl.BlockSpec((BM, UNITS), lambda i: (i, 0)),
        ],
        out_specs=pl.BlockSpec((BM, UNITS), lambda i: (i, 0)),
        out_shape=jax.ShapeDtypeStruct((BATCH * N, UNITS), jnp.float32),
    )(x5, Wc_r, bc.reshape(1, -1), hxr, u)


def kernel(inputs, hx, src0, dst0, w0, src1, dst1, w1, Wg, bg, Wc, bc):
    it = jnp.transpose(inputs.reshape(BATCH, N, INPUT_DIM), (1, 0, 2))
    hxt = jnp.transpose(hx.reshape(BATCH, N, UNITS), (1, 0, 2))
    hxr = hxt.reshape(BATCH * N, UNITS)
    x0 = jnp.concatenate([it, hxt], axis=2).reshape(N, ROW)

    Wg_r = Wg.reshape(IN_SIZE, NUM_MAT, 2 * UNITS).transpose(1, 0, 2).reshape(
        NUM_MAT * IN_SIZE, 2 * UNITS)
    Wc_r = Wc.reshape(IN_SIZE, NUM_MAT, UNITS).transpose(1, 0, 2).reshape(
        NUM_MAT * IN_SIZE, UNITS)

    ms = _diffusion(x0, src0, dst0, w0, src1, dst1, w1)
    x5 = jnp.concatenate(
        [m.reshape(BATCH * N, IN_SIZE) for m in ms], axis=1)
    state2, u = _gate_call(x5, Wg_r, bg, hxr)

    x0c = jnp.concatenate([it, state2.reshape(N, BATCH, UNITS)], axis=2).reshape(N, ROW)
    msc = _diffusion(x0c, src0, dst0, w0, src1, dst1, w1)
    x5c = jnp.concatenate(
        [m.reshape(BATCH * N, IN_SIZE) for m in msc], axis=1)
    new = _cand_call(x5c, Wc_r, bc, hxr, u)

    return jnp.transpose(new.reshape(N, BATCH, UNITS), (1, 0, 2)).reshape(
        BATCH, N * UNITS)
